# baseline (speedup 1.0000x reference)
import functools

import jax
import jax.numpy as jnp
from jax import lax
from jax.experimental import pallas as pl
from jax.experimental.pallas import tpu as pltpu
from jax.experimental.pallas import tpu_sc as plsc

_N = 10000
_E = 320000
_HID = 200
_HP = 256
_NP = 10240
_NG = _NP // 128
_NC = 2
_NS = 16
_NW = _NC * _NS
_EPW = _E // _NW
_L = 16

_MESH = plsc.VectorSubcoreMesh(
    core_axis_name="c", subcore_axis_name="s", num_cores=_NC, num_subcores=_NS
)

_TILES = _E // 128
_TPW = _TILES // _NW
_EC_BASE = _TPW * 128
_EC_MAX = _EC_BASE + 128


def _load_edges(edge_hbm, ev, wid):
    t0 = _TPW * wid + jnp.minimum(wid, 4)
    c0 = pl.multiple_of(t0 * 128, 128)

    @pl.when(wid < 4)
    def _():
        pltpu.sync_copy(edge_hbm.at[:, pl.ds(c0, _EC_MAX)], ev)

    @pl.when(wid >= 4)
    def _():
        pltpu.sync_copy(edge_hbm.at[:, pl.ds(c0, _EC_BASE)],
                        ev.at[:, pl.ds(0, _EC_BASE)])


def _worker_id():
    return lax.axis_index("s") * _NC + lax.axis_index("c")


_STRIPE = _NP // _NS


def _reduce_to_core(acc_v, parts_sm, red_v, out_hbm):
    cid = lax.axis_index("c")
    sid = lax.axis_index("s")
    pltpu.sync_copy(acc_v, parts_sm.at[sid])
    plsc.subcore_barrier()
    base = sid * _STRIPE
    pltpu.sync_copy(parts_sm.at[:, pl.ds(base, _STRIPE)], red_v)

    def body(i, carry):
        cbase = i * _L
        tot = red_v[0, pl.ds(cbase, _L)]
        for t in range(1, _NS):
            tot = tot + red_v[t, pl.ds(cbase, _L)]
        acc_v[pl.ds(base + cbase, _L)] = tot
        return carry

    lax.fori_loop(0, _STRIPE // _L, body, 0)
    pltpu.sync_copy(acc_v.at[pl.ds(base, _STRIPE)], out_hbm.at[cid, pl.ds(base, _STRIPE)])


def _zero_vmem(ref, n):
    zeros = jnp.zeros((_L,), jnp.float32)
    unroll = 8

    def body(i, carry):
        base = i * (unroll * _L)
        for k in range(unroll):
            ref[pl.ds(base + k * _L, _L)] = zeros
        return carry

    lax.fori_loop(0, n // (unroll * _L), body, 0)


@functools.partial(
    pl.kernel,
    out_type=jax.ShapeDtypeStruct((_NC, _NP), jnp.float32),
    mesh=_MESH,
    scratch_types=[
        pltpu.VMEM((2, _EC_MAX), jnp.int32),
        pltpu.VMEM((_NP,), jnp.float32),
        pltpu.VMEM((_NP,), jnp.float32),
        pltpu.VMEM_SHARED((_NS, _NP), jnp.float32),
        pltpu.VMEM((_NS, _STRIPE), jnp.float32),
    ],
    compiler_params=pltpu.CompilerParams(needs_layout_passes=False),
)
def _sc_degree(edge_hbm, out_hbm, ev, acc_v, acc2_v, parts_sm, red_v):
    wid = _worker_id()
    _load_edges(edge_hbm, ev, wid)
    _zero_vmem(acc_v, _NP)
    _zero_vmem(acc2_v, _NP)
    ones = jnp.ones((_L,), jnp.float32)

    @plsc.parallel_loop(0, _EC_BASE // (2 * _L), unroll=12)
    def _(i):
        idx = ev[1, pl.ds(2 * i * _L, _L)]
        plsc.addupdate_scatter(acc_v, [idx], ones)
        idx2 = ev[1, pl.ds((2 * i + 1) * _L, _L)]
        plsc.addupdate_scatter(acc2_v, [idx2], ones)

    @pl.when(wid < 4)
    def _():
        for k in range(_EC_BASE // _L, _EC_MAX // _L):
            idx = ev[1, pl.ds(k * _L, _L)]
            plsc.addupdate_scatter(acc_v, [idx], ones)

    @plsc.parallel_loop(0, _NP // _L, unroll=8)
    def _(i):
        acc_v[pl.ds(i * _L, _L)] = (acc_v[pl.ds(i * _L, _L)]
                                    + acc2_v[pl.ds(i * _L, _L)])

    _reduce_to_core(acc_v, parts_sm, red_v, out_hbm)


@functools.partial(
    pl.kernel,
    out_type=jax.ShapeDtypeStruct((_NC, _NP), jnp.float32),
    mesh=_MESH,
    scratch_types=[
        pltpu.VMEM((2, _EC_MAX), jnp.int32),
        pltpu.VMEM((_NP,), jnp.float32),
        pltpu.VMEM((_NP,), jnp.float32),
        pltpu.VMEM((_NP,), jnp.float32),
        pltpu.VMEM_SHARED((_NS, _NP), jnp.float32),
        pltpu.VMEM((_NS, _STRIPE), jnp.float32),
    ],
    compiler_params=pltpu.CompilerParams(needs_layout_passes=False),
)
def _sc_message(edge_hbm, u_hbm, out_hbm, ev, u_v, acc_v, acc2_v,
                parts_sm, red_v):
    wid = _worker_id()
    pltpu.sync_copy(u_hbm, u_v)
    _load_edges(edge_hbm, ev, wid)
    _zero_vmem(acc_v, _NP)
    _zero_vmem(acc2_v, _NP)

    @plsc.parallel_loop(0, _EC_BASE // (2 * _L), unroll=12)
    def _(i):
        sidx = ev[0, pl.ds(2 * i * _L, _L)]
        didx = ev[1, pl.ds(2 * i * _L, _L)]
        vals = plsc.load_gather(u_v, [sidx])
        plsc.addupdate_scatter(acc_v, [didx], vals)
        sidx2 = ev[0, pl.ds((2 * i + 1) * _L, _L)]
        didx2 = ev[1, pl.ds((2 * i + 1) * _L, _L)]
        vals2 = plsc.load_gather(u_v, [sidx2])
        plsc.addupdate_scatter(acc2_v, [didx2], vals2)

    @pl.when(wid < 4)
    def _():
        for k in range(_EC_BASE // _L, _EC_MAX // _L):
            sidx = ev[0, pl.ds(k * _L, _L)]
            didx = ev[1, pl.ds(k * _L, _L)]
            vals = plsc.load_gather(u_v, [sidx])
            plsc.addupdate_scatter(acc_v, [didx], vals)

    @plsc.parallel_loop(0, _NP // _L, unroll=8)
    def _(i):
        acc_v[pl.ds(i * _L, _L)] = (acc_v[pl.ds(i * _L, _L)]
                                    + acc2_v[pl.ds(i * _L, _L)])

    _reduce_to_core(acc_v, parts_sm, red_v, out_hbm)


def _matvec_body(x_ref, w_ref, h_ref):
    h_ref[...] = lax.dot(x_ref[...], w_ref[...], preferred_element_type=jnp.float32)


def _prep_body(degp_ref, h_ref, dinv_ref, u_ref):
    deg = jnp.sum(degp_ref[...], axis=0) + 1.0
    dinv = lax.rsqrt(deg)
    dinv_ref[...] = dinv
    u_ref[...] = dinv * h_ref[...]


_SLICES = 40


def _mlp_body(s0_ref, s1_ref, dinv_ref, u_ref, bg_ref, w1c_ref, bsT_ref,
              w2_ref, w3_ref, w4_ref, w5_ref, w6_ref, w7_ref, w8_ref,
              b8_ref, o_ref):
    z = dinv_ref[...] * (s0_ref[...] + s1_ref[...] + u_ref[...]) + bg_ref[0, 0]
    cols = [z[g:g + 1, :] * w1c_ref[...] for g in range(_SLICES)]
    aT = jnp.concatenate(cols, axis=1)
    aT = jnp.maximum(aT + bsT_ref[:, 0:1], 0.0)
    dn = (((0,), (0,)), ((), ()))
    for l, w_ref in enumerate((w2_ref, w3_ref, w4_ref, w5_ref, w6_ref, w7_ref),
                              start=1):
        aT = lax.dot_general(w_ref[...], aT.astype(jnp.bfloat16), dn,
                             preferred_element_type=jnp.float32)
        aT = jnp.maximum(aT + bsT_ref[:, l:l + 1], 0.0)
    logT = lax.dot_general(w8_ref[...], aT.astype(jnp.bfloat16), dn,
                           preferred_element_type=jnp.float32)
    o_ref[...] = jax.nn.sigmoid(logT + b8_ref[0, 0])


def kernel(x, edge_index, W_gcn, b_gcn, mlp_Ws, mlp_bs):
    f32 = jnp.float32

    x_pad = jnp.pad(x, ((0, _NP - _N), (0, 0)))

    h = pl.pallas_call(
        _matvec_body,
        out_shape=jax.ShapeDtypeStruct((_NP, 1), f32),
    )(x_pad, W_gcn)

    degp = _sc_degree(edge_index)

    dinv, u = pl.pallas_call(
        _prep_body,
        out_shape=(
            jax.ShapeDtypeStruct((_NG, 128), f32),
            jax.ShapeDtypeStruct((_NG, 128), f32),
        ),
    )(degp.reshape(_NC, _NG, 128), h.reshape(_NG, 128))

    sp = _sc_message(edge_index, u.reshape(_NP))

    bf16 = jnp.bfloat16
    w1c = mlp_Ws[0].reshape(_HID, 1)
    wmid = [w.astype(bf16) for w in mlp_Ws[1:7]]
    w8 = mlp_Ws[7].astype(bf16)
    bsT = jnp.stack(mlp_bs[:7], axis=1)
    b8 = mlp_bs[7].reshape(1, 1)

    rows = 128 * _SLICES
    grid = _NP // rows
    full = lambda shape: pl.BlockSpec(shape, lambda *_: tuple(0 for _ in shape))

    sp2 = sp.reshape(_NC * _NG, 128)
    operands = [sp2, sp2, dinv, u,
                b_gcn.reshape(1, 1), w1c, bsT] + wmid + [w8, b8]
    in_specs = [pl.BlockSpec((_SLICES, 128), lambda i: (i, 0)),
                pl.BlockSpec((_SLICES, 128), lambda i: (i + _NG // _SLICES, 0)),
                pl.BlockSpec((_SLICES, 128), lambda i: (i, 0)),
                pl.BlockSpec((_SLICES, 128), lambda i: (i, 0)),
                full((1, 1)), full((_HID, 1)), full((_HID, 7))]
    in_specs += [full((_HID, _HID))] * 6
    in_specs += [full((_HID, 1)), full((1, 1))]

    out = pl.pallas_call(
        _mlp_body,
        grid=(grid,),
        in_specs=in_specs,
        out_specs=pl.BlockSpec((1, rows), lambda i: (0, i)),
        out_shape=jax.ShapeDtypeStruct((1, _NP), f32),
    )(*operands)

    return out.reshape(_NP)[:_N, None]


# speedup vs baseline: 1.0964x; 1.0964x over previous
import functools

import jax
import jax.numpy as jnp
from jax import lax
from jax.experimental import pallas as pl
from jax.experimental.pallas import tpu as pltpu
from jax.experimental.pallas import tpu_sc as plsc

_N = 10000
_E = 320000
_HID = 200
_HP = 256
_NP = 10240
_NG = _NP // 128
_NC = 2
_NS = 16
_NW = _NC * _NS
_EPW = _E // _NW
_L = 16

_MESH = plsc.VectorSubcoreMesh(
    core_axis_name="c", subcore_axis_name="s", num_cores=_NC, num_subcores=_NS
)

_TILES = _E // 128
_TPW = _TILES // _NW
_EC_BASE = _TPW * 128
_EC_MAX = _EC_BASE + 128


def _load_edges(edge_hbm, ev, wid):
    t0 = _TPW * wid + jnp.minimum(wid, 4)
    c0 = pl.multiple_of(t0 * 128, 128)

    @pl.when(wid < 4)
    def _():
        pltpu.sync_copy(edge_hbm.at[:, pl.ds(c0, _EC_MAX)], ev)

    @pl.when(wid >= 4)
    def _():
        pltpu.sync_copy(edge_hbm.at[:, pl.ds(c0, _EC_BASE)],
                        ev.at[:, pl.ds(0, _EC_BASE)])


def _worker_id():
    return lax.axis_index("s") * _NC + lax.axis_index("c")


_STRIPE = _NP // _NS


def _reduce_to_core(acc_v, parts_sm, red_v, out_hbm):
    cid = lax.axis_index("c")
    sid = lax.axis_index("s")
    pltpu.sync_copy(acc_v, parts_sm.at[sid])
    plsc.subcore_barrier()
    base = sid * _STRIPE
    pltpu.sync_copy(parts_sm.at[:, pl.ds(base, _STRIPE)], red_v)

    def body(i, carry):
        cbase = i * _L
        tot = red_v[0, pl.ds(cbase, _L)]
        for t in range(1, _NS):
            tot = tot + red_v[t, pl.ds(cbase, _L)]
        acc_v[pl.ds(base + cbase, _L)] = tot
        return carry

    lax.fori_loop(0, _STRIPE // _L, body, 0)
    pltpu.sync_copy(acc_v.at[pl.ds(base, _STRIPE)], out_hbm.at[cid, pl.ds(base, _STRIPE)])


def _zero_vmem(ref, n):
    zeros = jnp.zeros((_L,), jnp.float32)
    unroll = 8

    def body(i, carry):
        base = i * (unroll * _L)
        for k in range(unroll):
            ref[pl.ds(base + k * _L, _L)] = zeros
        return carry

    lax.fori_loop(0, n // (unroll * _L), body, 0)


@functools.partial(
    pl.kernel,
    out_type=jax.ShapeDtypeStruct((_NC, _NP), jnp.float32),
    mesh=_MESH,
    scratch_types=[
        pltpu.VMEM((2, _EC_MAX), jnp.int32),
        pltpu.VMEM((_NP,), jnp.float32),
        pltpu.VMEM_SHARED((_NS, _NP), jnp.float32),
        pltpu.VMEM((_NS, _STRIPE), jnp.float32),
    ],
    compiler_params=pltpu.CompilerParams(needs_layout_passes=False),
)
def _sc_degree(edge_hbm, out_hbm, ev, acc_v, parts_sm, red_v):
    wid = _worker_id()
    _load_edges(edge_hbm, ev, wid)
    _zero_vmem(acc_v, _NP)
    ones = jnp.ones((_L,), jnp.float32)

    @plsc.parallel_loop(0, _EC_BASE // _L, unroll=24)
    def _(i):
        idx = ev[1, pl.ds(i * _L, _L)]
        plsc.addupdate_scatter(acc_v, [idx], ones)

    @pl.when(wid < 4)
    def _():
        for k in range(_EC_BASE // _L, _EC_MAX // _L):
            idx = ev[1, pl.ds(k * _L, _L)]
            plsc.addupdate_scatter(acc_v, [idx], ones)

    _reduce_to_core(acc_v, parts_sm, red_v, out_hbm)


@functools.partial(
    pl.kernel,
    out_type=jax.ShapeDtypeStruct((_NC, _NP), jnp.float32),
    mesh=_MESH,
    scratch_types=[
        pltpu.VMEM((2, _EC_MAX), jnp.int32),
        pltpu.VMEM((_NP,), jnp.float32),
        pltpu.VMEM((_NP,), jnp.float32),
        pltpu.VMEM_SHARED((_NS, _NP), jnp.float32),
        pltpu.VMEM((_NS, _STRIPE), jnp.float32),
    ],
    compiler_params=pltpu.CompilerParams(needs_layout_passes=False),
)
def _sc_message(edge_hbm, u_hbm, out_hbm, ev, u_v, acc_v,
                parts_sm, red_v):
    wid = _worker_id()
    pltpu.sync_copy(u_hbm, u_v)
    _load_edges(edge_hbm, ev, wid)
    _zero_vmem(acc_v, _NP)

    @plsc.parallel_loop(0, _EC_BASE // _L, unroll=24)
    def _(i):
        sidx = ev[0, pl.ds(i * _L, _L)]
        didx = ev[1, pl.ds(i * _L, _L)]
        vals = plsc.load_gather(u_v, [sidx])
        plsc.addupdate_scatter(acc_v, [didx], vals)

    @pl.when(wid < 4)
    def _():
        for k in range(_EC_BASE // _L, _EC_MAX // _L):
            sidx = ev[0, pl.ds(k * _L, _L)]
            didx = ev[1, pl.ds(k * _L, _L)]
            vals = plsc.load_gather(u_v, [sidx])
            plsc.addupdate_scatter(acc_v, [didx], vals)

    _reduce_to_core(acc_v, parts_sm, red_v, out_hbm)


def _matvec_body(x_ref, w_ref, h_ref):
    h_ref[...] = lax.dot_general(x_ref[...], w_ref[...],
                                 (((2,), (0,)), ((), ())),
                                 preferred_element_type=jnp.float32)


def _prep_body(degp_ref, h_ref, dinv_ref, u_ref):
    deg = jnp.sum(degp_ref[...], axis=0) + 1.0
    dinv = lax.rsqrt(deg)
    dinv_ref[...] = dinv
    u_ref[...] = dinv * h_ref[...]


_SLICES = 40


def _mlp_body(s0_ref, s1_ref, dinv_ref, u_ref, bg_ref, w1c_ref, bsT_ref,
              w2_ref, w3_ref, w4_ref, w5_ref, w6_ref, w7_ref, w8_ref,
              b8_ref, o_ref):
    z = dinv_ref[...] * (s0_ref[...] + s1_ref[...] + u_ref[...]) + bg_ref[0, 0]
    cols = [z[g:g + 1, :] * w1c_ref[...] for g in range(_SLICES)]
    aT = jnp.concatenate(cols, axis=1)
    aT = jnp.maximum(aT + bsT_ref[:, 0:1], 0.0)
    dn = (((0,), (0,)), ((), ()))
    for l, w_ref in enumerate((w2_ref, w3_ref, w4_ref, w5_ref, w6_ref, w7_ref),
                              start=1):
        aT = lax.dot_general(w_ref[...], aT.astype(jnp.bfloat16), dn,
                             preferred_element_type=jnp.float32)
        aT = jnp.maximum(aT + bsT_ref[:, l:l + 1], 0.0)
    logT = lax.dot_general(w8_ref[...], aT.astype(jnp.bfloat16), dn,
                           preferred_element_type=jnp.float32)
    o_ref[...] = jax.nn.sigmoid(logT + b8_ref[0, 0])


def kernel(x, edge_index, W_gcn, b_gcn, mlp_Ws, mlp_bs):
    f32 = jnp.float32

    x_pad = jnp.pad(x, ((0, _NP - _N), (0, 0)))

    h = pl.pallas_call(
        _matvec_body,
        out_shape=jax.ShapeDtypeStruct((_NG, 128), f32),
    )(x_pad.reshape(_NG, 128, 128), W_gcn.reshape(128))

    degp = _sc_degree(edge_index)

    dinv, u = pl.pallas_call(
        _prep_body,
        out_shape=(
            jax.ShapeDtypeStruct((_NG, 128), f32),
            jax.ShapeDtypeStruct((_NG, 128), f32),
        ),
    )(degp.reshape(_NC, _NG, 128), h)

    sp = _sc_message(edge_index, u.reshape(_NP))

    bf16 = jnp.bfloat16
    w1c = mlp_Ws[0].reshape(_HID, 1)
    wmid = [w.astype(bf16) for w in mlp_Ws[1:7]]
    w8 = mlp_Ws[7].astype(bf16)
    bsT = jnp.stack(mlp_bs[:7], axis=1)
    b8 = mlp_bs[7].reshape(1, 1)

    rows = 128 * _SLICES
    grid = _NP // rows
    full = lambda shape: pl.BlockSpec(shape, lambda *_: tuple(0 for _ in shape))

    sp2 = sp.reshape(_NC * _NG, 128)
    operands = [sp2, sp2, dinv, u,
                b_gcn.reshape(1, 1), w1c, bsT] + wmid + [w8, b8]
    in_specs = [pl.BlockSpec((_SLICES, 128), lambda i: (i, 0)),
                pl.BlockSpec((_SLICES, 128), lambda i: (i + _NG // _SLICES, 0)),
                pl.BlockSpec((_SLICES, 128), lambda i: (i, 0)),
                pl.BlockSpec((_SLICES, 128), lambda i: (i, 0)),
                full((1, 1)), full((_HID, 1)), full((_HID, 7))]
    in_specs += [full((_HID, _HID))] * 6
    in_specs += [full((_HID, 1)), full((1, 1))]

    out = pl.pallas_call(
        _mlp_body,
        grid=(grid,),
        in_specs=in_specs,
        out_specs=pl.BlockSpec((1, rows), lambda i: (0, i)),
        out_shape=jax.ShapeDtypeStruct((1, _NP), f32),
    )(*operands)

    return out.reshape(_NP)[:_N, None]
